# Initial kernel scaffold; baseline (speedup 1.0000x reference)
import jax
import jax.numpy as jnp
from jax.experimental import pallas as pl


def kernel(x, edge_index, W0, b0, W1, b1, W2, b2):
    raise NotImplementedError("write your pallas kernel here")


# speedup vs baseline: 1.0378x; 1.0378x over previous
import functools

import jax
import jax.numpy as jnp
from jax import lax
from jax.experimental import pallas as pl
from jax.experimental.pallas import tpu as pltpu
from jax.experimental.pallas import tpu_sc as plsc
import numpy as np

N = 10000
E = 320000
D = 128
DELTA = float(np.log(2.0))

NC = 2
NS = 16
NW = NC * NS
P = 2
R = 160
NPAD = NW * P * R
C = 512
G = 128
NG = C // 16
NCHUNK = E // C

_mesh = plsc.VectorSubcoreMesh(core_axis_name="c", subcore_axis_name="s")


def _worker_id():
    return lax.axis_index("s") * NC + lax.axis_index("c")


@functools.partial(
    pl.kernel,
    out_type=jax.ShapeDtypeStruct((NW * P, R, 16), jnp.float32),
    mesh=_mesh,
    scratch_types=[
        pltpu.VMEM((C,), jnp.int32),
        pltpu.VMEM((32,), jnp.int32),
        pltpu.VMEM((R, 16), jnp.float32),
    ],
    compiler_params=pltpu.CompilerParams(needs_layout_passes=False),
)
def _deg_kernel(src_hbm, deg_o, chunk_v, tmp_v, dacc_v):
    wid = _worker_id()
    zero16 = jnp.zeros((16,), jnp.float32)
    onehot = jnp.where(lax.iota(jnp.int32, 16) == 0, 1.0, 0.0)

    def pass_body(p, _):
        seg = p * NW + wid
        start = seg * R

        def zero_body(i, _):
            dacc_v[i] = zero16
            return 0

        lax.fori_loop(0, R, zero_body, 0)

        def chunk_body(c, _):
            pltpu.sync_copy(src_hbm.at[pl.ds(c * C, C)], chunk_v)

            def group_body(g, _):
                sv = chunk_v[pl.ds(g * 16, 16)]
                loc = sv - start
                m = (sv >= start) & (sv < start + R)
                cs = plsc.cumsum(m.astype(jnp.int32))
                pos = jnp.where(m, cs - 1, 0)
                plsc.store_scatter(tmp_v.at[pl.ds(0, 32)], [pos], loc, mask=m)
                k = cs[15]

                def edge_body(i, _):
                    d = tmp_v[pl.ds(i, 16)][0]
                    plsc.addupdate(dacc_v.at[d], onehot)
                    return 0

                lax.fori_loop(0, k, edge_body, 0)
                return 0

            lax.fori_loop(0, NG, group_body, 0)
            return 0

        lax.fori_loop(0, NCHUNK, chunk_body, 0)
        pltpu.sync_copy(dacc_v, deg_o.at[seg])
        return 0

    lax.fori_loop(0, P, pass_body, 0)


_agg_out_type = (
    jax.ShapeDtypeStruct((NPAD, D), jnp.float32),
    jax.ShapeDtypeStruct((NPAD, D), jnp.float32),
    jax.ShapeDtypeStruct((NPAD, D), jnp.float32),
    jax.ShapeDtypeStruct((NPAD, D), jnp.float32),
    jax.ShapeDtypeStruct((NW * P, R, 16), jnp.float32),
)


@functools.partial(
    pl.kernel,
    out_type=_agg_out_type,
    mesh=_mesh,
    scratch_types=[
        pltpu.VMEM((C,), jnp.int32),
        pltpu.VMEM((C,), jnp.int32),
        pltpu.VMEM((G + 32,), jnp.int32),
        pltpu.VMEM((G + 32,), jnp.int32),
        pltpu.VMEM((G,), jnp.int32),
        pltpu.VMEM((G, D), jnp.float32),
        pltpu.VMEM((R, D), jnp.float32),
        pltpu.VMEM((R, D), jnp.float32),
        pltpu.VMEM((R, D), jnp.float32),
        pltpu.VMEM((R, D), jnp.float32),
        pltpu.VMEM((R, 16), jnp.float32),
        pltpu.SemaphoreType.DMA,
    ],
    compiler_params=pltpu.CompilerParams(needs_layout_passes=False),
)
def _agg_kernel(h_hbm, src_hbm, dst_hbm, sum_o, sq_o, mx_o, mn_o, cnt_o,
                dst_v, srcc_v, idx_v, loc_v, gidx_v, rows_v,
                sacc, qacc, xacc, nacc, cacc, sem):
    wid = _worker_id()
    zero16 = jnp.zeros((16,), jnp.float32)
    ninf16 = jnp.full((16,), -jnp.inf, jnp.float32)
    pinf16 = jnp.full((16,), jnp.inf, jnp.float32)
    lane = lax.iota(jnp.int32, 16)
    onehot = jnp.where(lane == 0, 1.0, 0.0)

    def process(nproc):
        for i in range(G // 16):
            gidx_v[pl.ds(16 * i, 16)] = idx_v[pl.ds(16 * i, 16)]
        pltpu.async_copy(h_hbm.at[gidx_v], rows_v, sem).wait()

        def edge_body(e, _):
            d = loc_v[pl.ds(e, 16)][0]
            plsc.addupdate(cacc.at[d], onehot)
            for j in range(D // 16):
                sl = pl.ds(16 * j, 16)
                m = rows_v[e, sl]
                plsc.addupdate(sacc.at[d, sl], m)
                plsc.addupdate(qacc.at[d, sl], m * m)
                xacc[d, sl] = jnp.maximum(xacc[d, sl], m)
                nacc[d, sl] = jnp.minimum(nacc[d, sl], m)
            return 0

        lax.fori_loop(0, nproc, edge_body, 0)

    def pass_body(p, _):
        seg = p * NW + wid
        start = seg * R

        def zero_body(i, _):
            for j in range(D // 16):
                sl = pl.ds(16 * j, 16)
                sacc[i, sl] = zero16
                qacc[i, sl] = zero16
                xacc[i, sl] = ninf16
                nacc[i, sl] = pinf16
            cacc[i] = zero16
            return 0

        lax.fori_loop(0, R, zero_body, 0)
        for i in range((G + 32) // 16):
            idx_v[pl.ds(16 * i, 16)] = jnp.zeros((16,), jnp.int32)

        def chunk_body(c, pend):
            pltpu.sync_copy(dst_hbm.at[pl.ds(c * C, C)], dst_v)
            pltpu.sync_copy(src_hbm.at[pl.ds(c * C, C)], srcc_v)

            def group_body(g, pend):
                dv = dst_v[pl.ds(g * 16, 16)]
                sv = srcc_v[pl.ds(g * 16, 16)]
                m = (dv >= start) & (dv < start + R)
                cs = plsc.cumsum(m.astype(jnp.int32))
                pos = jnp.where(m, pend + cs - 1, 0)
                plsc.store_scatter(idx_v.at[pl.ds(0, G + 32)], [pos], sv,
                                   mask=m)
                plsc.store_scatter(loc_v.at[pl.ds(0, G + 32)], [pos],
                                   dv - start, mask=m)
                pend = pend + cs[15]

                def flush(pend):
                    process(G)
                    rem = pend - G
                    tmask = lane < rem
                    ti = idx_v[pl.ds(G, 16)]
                    tl = loc_v[pl.ds(G, 16)]
                    plsc.store_scatter(idx_v.at[pl.ds(0, G + 32)], [lane],
                                       ti, mask=tmask)
                    plsc.store_scatter(loc_v.at[pl.ds(0, G + 32)], [lane],
                                       tl, mask=tmask)
                    return rem

                return lax.cond(pend >= G, flush, lambda q: q, pend)

            return lax.fori_loop(0, NG, group_body, pend)

        pend = lax.fori_loop(0, NCHUNK, chunk_body, 0)

        def tail(pend):
            process(pend)
            return 0

        lax.cond(pend > 0, tail, lambda q: 0, pend)

        row0 = pl.ds(seg * R, R)
        pltpu.sync_copy(sacc, sum_o.at[row0])
        pltpu.sync_copy(qacc, sq_o.at[row0])
        pltpu.sync_copy(xacc, mx_o.at[row0])
        pltpu.sync_copy(nacc, mn_o.at[row0])
        pltpu.sync_copy(cacc, cnt_o.at[seg])
        return 0

    lax.fori_loop(0, P, pass_body, 0)


RB = 1000


def _tc_body(sum_r, sq_r, mx_r, mn_r, cnt_r, deg_r, h_r, w_r, b_r, out_r):
    cnt = cnt_r[...]
    cntc = jnp.maximum(cnt, 1.0)
    s = sum_r[...]
    mean = s / cntc
    var = jnp.maximum(sq_r[...] / cntc - mean * mean, 0.0)
    std = jnp.sqrt(var + 1e-5)
    pos = cnt > 0.0
    mx = jnp.where(pos, mx_r[...], 0.0)
    mn = jnp.where(pos, mn_r[...], 0.0)
    agg = jnp.concatenate([mean, mx, mn, std], axis=1)
    wa = w_r[0:4 * D, :]
    wb = w_r[4 * D:8 * D, :]
    wc = w_r[8 * D:12 * D, :]
    b1 = jnp.dot(agg, wa, preferred_element_type=jnp.float32)
    b2 = jnp.dot(agg, wb, preferred_element_type=jnp.float32)
    b3 = jnp.dot(agg, wc, preferred_element_type=jnp.float32)
    degc = jnp.maximum(deg_r[...], 1.0)
    logd = jnp.log(degc + 1.0)
    amp = logd * (1.0 / DELTA)
    att = DELTA / logd
    out_r[...] = b1 + amp * b2 + att * b3 + b_r[...] + h_r[...]


def _tc_layer(sum_a, sq_a, mx_a, mn_a, cnt_a, deg_a, h, w, b):
    grid = (N // RB,)
    row_spec = pl.BlockSpec((RB, D), lambda i: (i, 0))
    col_spec = pl.BlockSpec((RB, 1), lambda i: (i, 0))
    return pl.pallas_call(
        _tc_body,
        grid=grid,
        in_specs=[
            row_spec, row_spec, row_spec, row_spec,
            col_spec, col_spec, row_spec,
            pl.BlockSpec((12 * D, D), lambda i: (0, 0)),
            pl.BlockSpec((1, D), lambda i: (0, 0)),
        ],
        out_specs=row_spec,
        out_shape=jax.ShapeDtypeStruct((N, D), jnp.float32),
    )(sum_a, sq_a, mx_a, mn_a, cnt_a, deg_a, h, w, b)


def kernel(x, edge_index, W0, b0, W1, b1, W2, b2):
    src = edge_index[0]
    dst = edge_index[1]
    deg_o = _deg_kernel(src)
    deg = deg_o.reshape(NPAD, 16)[:N, 0:1]
    h = x
    for w, b in ((W0, b0), (W1, b1), (W2, b2)):
        sum_o, sq_o, mx_o, mn_o, cnt_o = _agg_kernel(h, src, dst)
        cnt = cnt_o.reshape(NPAD, 16)[:N, 0:1]
        h = _tc_layer(sum_o, sq_o, mx_o, mn_o, cnt, deg, h,
                      w, b.reshape(1, D))
    return h


# speedup vs baseline: 1.9673x; 1.8956x over previous
import functools

import jax
import jax.numpy as jnp
from jax import lax
from jax.experimental import pallas as pl
from jax.experimental.pallas import tpu as pltpu
from jax.experimental.pallas import tpu_sc as plsc
import numpy as np

N = 10000
E = 320000
D = 128
DELTA = float(np.log(2.0))

NC = 2
NS = 16
NW = NC * NS
P = 2
R = 160
NSEG = NW * P
NPAD = NSEG * R
C = 512
G = 128
NG = C // 16
NCHUNK = E // C
SEGCAP = E + G

_mesh = plsc.VectorSubcoreMesh(core_axis_name="c", subcore_axis_name="s")
_sc_params = pltpu.CompilerParams(needs_layout_passes=False)


def _worker_id():
    return lax.axis_index("s") * NC + lax.axis_index("c")


_build_out_type = (
    jax.ShapeDtypeStruct((NSEG * SEGCAP,), jnp.int32),
    jax.ShapeDtypeStruct((NSEG * SEGCAP,), jnp.int32),
    jax.ShapeDtypeStruct((NSEG * 16,), jnp.int32),
    jax.ShapeDtypeStruct((NSEG, R, 16), jnp.float32),
)


@functools.partial(
    pl.kernel,
    out_type=_build_out_type,
    mesh=_mesh,
    scratch_types=[
        pltpu.VMEM((C,), jnp.int32),
        pltpu.VMEM((C,), jnp.int32),
        pltpu.VMEM((G + 32,), jnp.int32),
        pltpu.VMEM((G + 32,), jnp.int32),
        pltpu.VMEM((32,), jnp.int32),
        pltpu.VMEM((16,), jnp.int32),
        pltpu.VMEM((R, 16), jnp.float32),
    ],
    compiler_params=_sc_params,
)
def _build_kernel(src_hbm, dst_hbm, esrc_o, eloc_o, ecnt_o, deg_o,
                  dst_v, srcc_v, idx_v, loc_v, tmp_v, cntb_v, dacc_v):
    wid = _worker_id()
    zero16 = jnp.zeros((16,), jnp.float32)
    lane = lax.iota(jnp.int32, 16)
    onehot = jnp.where(lane == 0, 1.0, 0.0)

    def pass_body(p, _):
        seg = p * NW + wid
        start = seg * R
        base = seg * SEGCAP

        def zero_body(i, _):
            dacc_v[i] = zero16
            return 0

        lax.fori_loop(0, R, zero_body, 0)

        def chunk_body(c, carry):
            pltpu.sync_copy(dst_hbm.at[pl.ds(c * C, C)], dst_v)
            pltpu.sync_copy(src_hbm.at[pl.ds(c * C, C)], srcc_v)

            def group_body(g, carry):
                pend, nfl = carry
                sv = srcc_v[pl.ds(g * 16, 16)]
                dv = dst_v[pl.ds(g * 16, 16)]

                ms = (sv >= start) & (sv < start + R)
                css = plsc.cumsum(ms.astype(jnp.int32))
                poss = jnp.where(ms, css - 1, 0)
                plsc.store_scatter(tmp_v.at[pl.ds(0, 32)], [poss],
                                   sv - start, mask=ms)
                ks = css[15]

                def sdeg_body(i, _):
                    d = tmp_v[pl.ds(i, 16)][0]
                    plsc.addupdate(dacc_v.at[d], onehot)
                    return 0

                lax.fori_loop(0, ks, sdeg_body, 0)

                m = (dv >= start) & (dv < start + R)
                cs = plsc.cumsum(m.astype(jnp.int32))
                pos = jnp.where(m, pend + cs - 1, 0)
                plsc.store_scatter(idx_v.at[pl.ds(0, G + 32)], [pos], sv,
                                   mask=m)
                plsc.store_scatter(loc_v.at[pl.ds(0, G + 32)], [pos],
                                   dv - start, mask=m)
                pend = pend + cs[15]

                def flush(carry):
                    pend, nfl = carry
                    off = base + nfl * G
                    pltpu.sync_copy(idx_v.at[pl.ds(0, G)],
                                    esrc_o.at[pl.ds(off, G)])
                    pltpu.sync_copy(loc_v.at[pl.ds(0, G)],
                                    eloc_o.at[pl.ds(off, G)])
                    rem = pend - G
                    tmask = lane < rem
                    ti = idx_v[pl.ds(G, 16)]
                    tl = loc_v[pl.ds(G, 16)]
                    plsc.store_scatter(idx_v.at[pl.ds(0, G + 32)], [lane],
                                       ti, mask=tmask)
                    plsc.store_scatter(loc_v.at[pl.ds(0, G + 32)], [lane],
                                       tl, mask=tmask)
                    return (rem, nfl + 1)

                return lax.cond(pend >= G, flush, lambda q: q, (pend, nfl))

            return lax.fori_loop(0, NG, group_body, carry)

        pend, nfl = lax.fori_loop(0, NCHUNK, chunk_body, (0, 0))

        def tail(carry):
            pend, nfl = carry
            for i in range(G // 16):
                sl = pl.ds(16 * i, 16)
                mm = (lane + 16 * i) >= pend
                loc_v[sl] = jnp.where(mm, R, loc_v[sl])
            off = base + nfl * G
            pltpu.sync_copy(idx_v.at[pl.ds(0, G)], esrc_o.at[pl.ds(off, G)])
            pltpu.sync_copy(loc_v.at[pl.ds(0, G)], eloc_o.at[pl.ds(off, G)])
            return (0, nfl + 1)

        _, nfl = lax.cond(pend > 0, tail, lambda q: q, (pend, nfl))

        cntb_v[pl.ds(0, 16)] = jnp.full((16,), 1, jnp.int32) * nfl
        pltpu.sync_copy(cntb_v, ecnt_o.at[pl.ds(seg * 16, 16)])
        pltpu.sync_copy(dacc_v, deg_o.at[seg])
        return 0

    lax.fori_loop(0, P, pass_body, 0)


_agg_out_type = (
    jax.ShapeDtypeStruct((NPAD, D), jnp.float32),
    jax.ShapeDtypeStruct((NPAD, D), jnp.float32),
    jax.ShapeDtypeStruct((NPAD, D), jnp.float32),
    jax.ShapeDtypeStruct((NPAD, D), jnp.float32),
    jax.ShapeDtypeStruct((NSEG, R, 16), jnp.float32),
)


@functools.partial(
    pl.kernel,
    out_type=_agg_out_type,
    mesh=_mesh,
    scratch_types=[
        pltpu.VMEM((G,), jnp.int32),
        pltpu.VMEM((G + 16,), jnp.int32),
        pltpu.VMEM((G, D), jnp.float32),
        pltpu.VMEM((R + 1, D), jnp.float32),
        pltpu.VMEM((R + 1, D), jnp.float32),
        pltpu.VMEM((R + 1, D), jnp.float32),
        pltpu.VMEM((R + 1, D), jnp.float32),
        pltpu.VMEM((R + 1, 16), jnp.float32),
        pltpu.VMEM((16,), jnp.int32),
        pltpu.SemaphoreType.DMA,
    ],
    compiler_params=_sc_params,
)
def _agg_kernel(h_hbm, esrc_hbm, eloc_hbm, ecnt_hbm,
                sum_o, sq_o, mx_o, mn_o, cnt_o,
                gidx_v, loc_v, rows_v, sacc, qacc, xacc, nacc, cacc,
                cnt_s, sem):
    wid = _worker_id()
    zero16 = jnp.zeros((16,), jnp.float32)
    ninf16 = jnp.full((16,), -jnp.inf, jnp.float32)
    pinf16 = jnp.full((16,), jnp.inf, jnp.float32)
    lane = lax.iota(jnp.int32, 16)
    onehot = jnp.where(lane == 0, 1.0, 0.0)

    def pass_body(p, _):
        seg = p * NW + wid
        base = seg * SEGCAP
        pltpu.sync_copy(ecnt_hbm.at[pl.ds(seg * 16, 16)], cnt_s)
        ng = cnt_s[pl.ds(0, 16)][0]

        def zero_body(i, _):
            for j in range(D // 16):
                sl = pl.ds(16 * j, 16)
                sacc[i, sl] = zero16
                qacc[i, sl] = zero16
                xacc[i, sl] = ninf16
                nacc[i, sl] = pinf16
            cacc[i] = zero16
            return 0

        lax.fori_loop(0, R + 1, zero_body, 0)

        def block_body(g, _):
            off = base + g * G
            pltpu.sync_copy(esrc_hbm.at[pl.ds(off, G)], gidx_v)
            pltpu.sync_copy(eloc_hbm.at[pl.ds(off, G)],
                            loc_v.at[pl.ds(0, G)])
            pltpu.async_copy(h_hbm.at[gidx_v], rows_v, sem).wait()

            def edge_body(e, _):
                d = loc_v[pl.ds(e, 16)][0]
                plsc.addupdate(cacc.at[d], onehot)
                for j in range(D // 16):
                    sl = pl.ds(16 * j, 16)
                    m = rows_v[e, sl]
                    plsc.addupdate(sacc.at[d, sl], m)
                    plsc.addupdate(qacc.at[d, sl], m * m)
                    xacc[d, sl] = jnp.maximum(xacc[d, sl], m)
                    nacc[d, sl] = jnp.minimum(nacc[d, sl], m)
                return 0

            lax.fori_loop(0, G, edge_body, 0)
            return 0

        lax.fori_loop(0, ng, block_body, 0)

        row0 = pl.ds(seg * R, R)
        pltpu.sync_copy(sacc.at[pl.ds(0, R)], sum_o.at[row0])
        pltpu.sync_copy(qacc.at[pl.ds(0, R)], sq_o.at[row0])
        pltpu.sync_copy(xacc.at[pl.ds(0, R)], mx_o.at[row0])
        pltpu.sync_copy(nacc.at[pl.ds(0, R)], mn_o.at[row0])
        pltpu.sync_copy(cacc.at[pl.ds(0, R)], cnt_o.at[seg])
        return 0

    lax.fori_loop(0, P, pass_body, 0)


RB = 1000


def _tc_body(sum_r, sq_r, mx_r, mn_r, cnt_r, deg_r, h_r, w_r, b_r, out_r):
    cnt = cnt_r[...]
    cntc = jnp.maximum(cnt, 1.0)
    s = sum_r[...]
    mean = s / cntc
    var = jnp.maximum(sq_r[...] / cntc - mean * mean, 0.0)
    std = jnp.sqrt(var + 1e-5)
    pos = cnt > 0.0
    mx = jnp.where(pos, mx_r[...], 0.0)
    mn = jnp.where(pos, mn_r[...], 0.0)
    agg = jnp.concatenate([mean, mx, mn, std], axis=1)
    wa = w_r[0:4 * D, :]
    wb = w_r[4 * D:8 * D, :]
    wc = w_r[8 * D:12 * D, :]
    b1 = jnp.dot(agg, wa, preferred_element_type=jnp.float32)
    b2 = jnp.dot(agg, wb, preferred_element_type=jnp.float32)
    b3 = jnp.dot(agg, wc, preferred_element_type=jnp.float32)
    degc = jnp.maximum(deg_r[...], 1.0)
    logd = jnp.log(degc + 1.0)
    amp = logd * (1.0 / DELTA)
    att = DELTA / logd
    out_r[...] = b1 + amp * b2 + att * b3 + b_r[...] + h_r[...]


def _tc_layer(sum_a, sq_a, mx_a, mn_a, cnt_a, deg_a, h, w, b):
    grid = (N // RB,)
    row_spec = pl.BlockSpec((RB, D), lambda i: (i, 0))
    col_spec = pl.BlockSpec((RB, 1), lambda i: (i, 0))
    return pl.pallas_call(
        _tc_body,
        grid=grid,
        in_specs=[
            row_spec, row_spec, row_spec, row_spec,
            col_spec, col_spec, row_spec,
            pl.BlockSpec((12 * D, D), lambda i: (0, 0)),
            pl.BlockSpec((1, D), lambda i: (0, 0)),
        ],
        out_specs=row_spec,
        out_shape=jax.ShapeDtypeStruct((N, D), jnp.float32),
    )(sum_a, sq_a, mx_a, mn_a, cnt_a, deg_a, h, w, b)


def kernel(x, edge_index, W0, b0, W1, b1, W2, b2):
    src = edge_index[0]
    dst = edge_index[1]
    esrc, eloc, ecnt, deg_o = _build_kernel(src, dst)
    deg = deg_o.reshape(NPAD, 16)[:N, 0:1]
    h = x
    for w, b in ((W0, b0), (W1, b1), (W2, b2)):
        sum_o, sq_o, mx_o, mn_o, cnt_o = _agg_kernel(h, esrc, eloc, ecnt)
        cnt = cnt_o.reshape(NPAD, 16)[:N, 0:1]
        h = _tc_layer(sum_o, sq_o, mx_o, mn_o, cnt, deg, h,
                      w, b.reshape(1, D))
    return h


# speedup vs baseline: 2.4779x; 1.2596x over previous
import functools

import jax
import jax.numpy as jnp
from jax import lax
from jax.experimental import pallas as pl
from jax.experimental.pallas import tpu as pltpu
from jax.experimental.pallas import tpu_sc as plsc
import numpy as np

N = 10000
E = 320000
D = 128
DELTA = float(np.log(2.0))

NC = 2
NS = 16
NW = NC * NS
R = 160
RW = 2 * R
NSEG = 2 * NW
NPAD = NSEG * R
C = 512
G = 128
NG = C // 16
NCHUNK = E // C
SEGCAP = E + G

_mesh = plsc.VectorSubcoreMesh(core_axis_name="c", subcore_axis_name="s")
_sc_params = pltpu.CompilerParams(needs_layout_passes=False)


def _worker_id():
    return lax.axis_index("s") * NC + lax.axis_index("c")


_build_out_type = (
    jax.ShapeDtypeStruct((NSEG * SEGCAP,), jnp.int32),
    jax.ShapeDtypeStruct((NSEG * SEGCAP,), jnp.int32),
    jax.ShapeDtypeStruct((NSEG * 16,), jnp.int32),
    jax.ShapeDtypeStruct((NW, RW, 16), jnp.float32),
)


@functools.partial(
    pl.kernel,
    out_type=_build_out_type,
    mesh=_mesh,
    scratch_types=[
        pltpu.VMEM((C,), jnp.int32),
        pltpu.VMEM((C,), jnp.int32),
        pltpu.VMEM((G + 32,), jnp.int32),
        pltpu.VMEM((G + 32,), jnp.int32),
        pltpu.VMEM((G + 32,), jnp.int32),
        pltpu.VMEM((G + 32,), jnp.int32),
        pltpu.VMEM((32,), jnp.int32),
        pltpu.VMEM((16,), jnp.int32),
        pltpu.VMEM((RW, 16), jnp.float32),
    ],
    compiler_params=_sc_params,
)
def _build_kernel(src_hbm, dst_hbm, esrc_o, eloc_o, ecnt_o, deg_o,
                  dst_v, srcc_v, ilo_v, llo_v, ihi_v, lhi_v,
                  tmp_v, cntb_v, dacc_v):
    wid = _worker_id()
    zero16 = jnp.zeros((16,), jnp.float32)
    lane = lax.iota(jnp.int32, 16)
    onehot = jnp.where(lane == 0, 1.0, 0.0)
    start = wid * RW

    def zero_body(i, _):
        dacc_v[i] = zero16
        return 0

    lax.fori_loop(0, RW, zero_body, 0)

    def append_and_flush(iv, lv, sv, locv, m, pend, nfl, seg):
        cs = plsc.cumsum(m.astype(jnp.int32))
        pos = jnp.where(m, pend + cs - 1, 0)
        plsc.store_scatter(iv.at[pl.ds(0, G + 32)], [pos], sv, mask=m)
        plsc.store_scatter(lv.at[pl.ds(0, G + 32)], [pos], locv, mask=m)
        pend = pend + cs[15]

        def flush(carry):
            pend, nfl = carry
            off = seg * SEGCAP + nfl * G
            pltpu.sync_copy(iv.at[pl.ds(0, G)], esrc_o.at[pl.ds(off, G)])
            pltpu.sync_copy(lv.at[pl.ds(0, G)], eloc_o.at[pl.ds(off, G)])
            rem = pend - G
            tmask = lane < rem
            ti = iv[pl.ds(G, 16)]
            tl = lv[pl.ds(G, 16)]
            plsc.store_scatter(iv.at[pl.ds(0, G + 32)], [lane], ti,
                               mask=tmask)
            plsc.store_scatter(lv.at[pl.ds(0, G + 32)], [lane], tl,
                               mask=tmask)
            return (rem, nfl + 1)

        return lax.cond(pend >= G, flush, lambda q: q, (pend, nfl))

    def chunk_body(c, carry):
        pltpu.sync_copy(dst_hbm.at[pl.ds(c * C, C)], dst_v)
        pltpu.sync_copy(src_hbm.at[pl.ds(c * C, C)], srcc_v)

        def group_body(g, carry):
            plo, nlo, phi, nhi = carry
            sv = srcc_v[pl.ds(g * 16, 16)]
            dv = dst_v[pl.ds(g * 16, 16)]

            ms = (sv >= start) & (sv < start + RW)
            css = plsc.cumsum(ms.astype(jnp.int32))
            poss = jnp.where(ms, css - 1, 0)
            plsc.store_scatter(tmp_v.at[pl.ds(0, 32)], [poss],
                               sv - start, mask=ms)
            ks = css[15]

            def sdeg_body(i, _):
                d = tmp_v[pl.ds(i, 16)][0]
                plsc.addupdate(dacc_v.at[d], onehot)
                return 0

            lax.fori_loop(0, ks, sdeg_body, 0)

            loc = dv - start
            md = (dv >= start) & (dv < start + RW)
            mlo = md & (loc < R)
            mhi = md & (loc >= R)
            plo, nlo = append_and_flush(ilo_v, llo_v, sv, loc, mlo,
                                        plo, nlo, 2 * wid)
            phi, nhi = append_and_flush(ihi_v, lhi_v, sv, loc - R, mhi,
                                        phi, nhi, 2 * wid + 1)
            return (plo, nlo, phi, nhi)

        return lax.fori_loop(0, NG, group_body, carry)

    plo, nlo, phi, nhi = lax.fori_loop(0, NCHUNK, chunk_body, (0, 0, 0, 0))

    def make_tail(iv, lv, seg):
        def tail(carry):
            pend, nfl = carry
            for i in range(G // 16):
                sl = pl.ds(16 * i, 16)
                mm = (lane + 16 * i) >= pend
                lv[sl] = jnp.where(mm, R, lv[sl])
            off = seg * SEGCAP + nfl * G
            pltpu.sync_copy(iv.at[pl.ds(0, G)], esrc_o.at[pl.ds(off, G)])
            pltpu.sync_copy(lv.at[pl.ds(0, G)], eloc_o.at[pl.ds(off, G)])
            return (0, nfl + 1)

        return tail

    _, nlo = lax.cond(plo > 0, make_tail(ilo_v, llo_v, 2 * wid),
                      lambda q: q, (plo, nlo))
    _, nhi = lax.cond(phi > 0, make_tail(ihi_v, lhi_v, 2 * wid + 1),
                      lambda q: q, (phi, nhi))

    cntb_v[pl.ds(0, 16)] = jnp.full((16,), 1, jnp.int32) * nlo
    pltpu.sync_copy(cntb_v, ecnt_o.at[pl.ds(2 * wid * 16, 16)])
    cntb_v[pl.ds(0, 16)] = jnp.full((16,), 1, jnp.int32) * nhi
    pltpu.sync_copy(cntb_v, ecnt_o.at[pl.ds((2 * wid + 1) * 16, 16)])
    pltpu.sync_copy(dacc_v, deg_o.at[wid])


_agg_out_type = (
    jax.ShapeDtypeStruct((NPAD, D), jnp.float32),
    jax.ShapeDtypeStruct((NPAD, D), jnp.float32),
    jax.ShapeDtypeStruct((NPAD, D), jnp.float32),
    jax.ShapeDtypeStruct((NPAD, D), jnp.float32),
    jax.ShapeDtypeStruct((NSEG, R, 16), jnp.float32),
)


@functools.partial(
    pl.kernel,
    out_type=_agg_out_type,
    mesh=_mesh,
    scratch_types=[
        pltpu.VMEM((G,), jnp.int32),
        pltpu.VMEM((G + 16,), jnp.int32),
        pltpu.VMEM((G, D), jnp.float32),
        pltpu.VMEM((R + 1, D), jnp.float32),
        pltpu.VMEM((R + 1, D), jnp.float32),
        pltpu.VMEM((R + 1, D), jnp.float32),
        pltpu.VMEM((R + 1, D), jnp.float32),
        pltpu.VMEM((R + 1, 16), jnp.float32),
        pltpu.VMEM((16,), jnp.int32),
        pltpu.SemaphoreType.DMA,
    ],
    compiler_params=_sc_params,
)
def _agg_kernel(h_hbm, esrc_hbm, eloc_hbm, ecnt_hbm,
                sum_o, sq_o, mx_o, mn_o, cnt_o,
                gidx_v, loc_v, rows_v, sacc, qacc, xacc, nacc, cacc,
                cnt_s, sem):
    wid = _worker_id()
    zero16 = jnp.zeros((16,), jnp.float32)
    ninf16 = jnp.full((16,), -jnp.inf, jnp.float32)
    pinf16 = jnp.full((16,), jnp.inf, jnp.float32)
    lane = lax.iota(jnp.int32, 16)
    onehot = jnp.where(lane == 0, 1.0, 0.0)

    def seg_body(sp, _):
        seg = 2 * wid + sp
        base = seg * SEGCAP
        pltpu.sync_copy(ecnt_hbm.at[pl.ds(seg * 16, 16)], cnt_s)
        ng = cnt_s[pl.ds(0, 16)][0]

        def zero_body(i, _):
            for j in range(D // 16):
                sl = pl.ds(16 * j, 16)
                sacc[i, sl] = zero16
                qacc[i, sl] = zero16
                xacc[i, sl] = ninf16
                nacc[i, sl] = pinf16
            cacc[i] = zero16
            return 0

        lax.fori_loop(0, R + 1, zero_body, 0)

        def block_body(g, _):
            off = base + g * G
            pltpu.sync_copy(esrc_hbm.at[pl.ds(off, G)], gidx_v)
            pltpu.sync_copy(eloc_hbm.at[pl.ds(off, G)],
                            loc_v.at[pl.ds(0, G)])
            pltpu.async_copy(h_hbm.at[gidx_v], rows_v, sem).wait()

            def edge_body(e, _):
                d = loc_v[pl.ds(e, 16)][0]
                plsc.addupdate(cacc.at[d], onehot)
                for j in range(D // 16):
                    sl = pl.ds(16 * j, 16)
                    m = rows_v[e, sl]
                    plsc.addupdate(sacc.at[d, sl], m)
                    plsc.addupdate(qacc.at[d, sl], m * m)
                    xacc[d, sl] = jnp.maximum(xacc[d, sl], m)
                    nacc[d, sl] = jnp.minimum(nacc[d, sl], m)
                return 0

            lax.fori_loop(0, G, edge_body, 0)
            return 0

        lax.fori_loop(0, ng, block_body, 0)

        row0 = pl.ds(seg * R, R)
        pltpu.sync_copy(sacc.at[pl.ds(0, R)], sum_o.at[row0])
        pltpu.sync_copy(qacc.at[pl.ds(0, R)], sq_o.at[row0])
        pltpu.sync_copy(xacc.at[pl.ds(0, R)], mx_o.at[row0])
        pltpu.sync_copy(nacc.at[pl.ds(0, R)], mn_o.at[row0])
        pltpu.sync_copy(cacc.at[pl.ds(0, R)], cnt_o.at[seg])
        return 0

    lax.fori_loop(0, 2, seg_body, 0)


RB = 1000


def _tc_body(sum_r, sq_r, mx_r, mn_r, cnt_r, deg_r, h_r, w_r, b_r, out_r):
    cnt = cnt_r[...]
    cntc = jnp.maximum(cnt, 1.0)
    s = sum_r[...]
    mean = s / cntc
    var = jnp.maximum(sq_r[...] / cntc - mean * mean, 0.0)
    std = jnp.sqrt(var + 1e-5)
    pos = cnt > 0.0
    mx = jnp.where(pos, mx_r[...], 0.0)
    mn = jnp.where(pos, mn_r[...], 0.0)
    agg = jnp.concatenate([mean, mx, mn, std], axis=1)
    wa = w_r[0:4 * D, :]
    wb = w_r[4 * D:8 * D, :]
    wc = w_r[8 * D:12 * D, :]
    b1 = jnp.dot(agg, wa, preferred_element_type=jnp.float32)
    b2 = jnp.dot(agg, wb, preferred_element_type=jnp.float32)
    b3 = jnp.dot(agg, wc, preferred_element_type=jnp.float32)
    degc = jnp.maximum(deg_r[...], 1.0)
    logd = jnp.log(degc + 1.0)
    amp = logd * (1.0 / DELTA)
    att = DELTA / logd
    out_r[...] = b1 + amp * b2 + att * b3 + b_r[...] + h_r[...]


def _tc_layer(sum_a, sq_a, mx_a, mn_a, cnt_a, deg_a, h, w, b):
    grid = (N // RB,)
    row_spec = pl.BlockSpec((RB, D), lambda i: (i, 0))
    col_spec = pl.BlockSpec((RB, 1), lambda i: (i, 0))
    return pl.pallas_call(
        _tc_body,
        grid=grid,
        in_specs=[
            row_spec, row_spec, row_spec, row_spec,
            col_spec, col_spec, row_spec,
            pl.BlockSpec((12 * D, D), lambda i: (0, 0)),
            pl.BlockSpec((1, D), lambda i: (0, 0)),
        ],
        out_specs=row_spec,
        out_shape=jax.ShapeDtypeStruct((N, D), jnp.float32),
    )(sum_a, sq_a, mx_a, mn_a, cnt_a, deg_a, h, w, b)


def kernel(x, edge_index, W0, b0, W1, b1, W2, b2):
    src = edge_index[0]
    dst = edge_index[1]
    esrc, eloc, ecnt, deg_o = _build_kernel(src, dst)
    deg = deg_o.reshape(NPAD, 16)[:N, 0:1]
    h = x
    for w, b in ((W0, b0), (W1, b1), (W2, b2)):
        sum_o, sq_o, mx_o, mn_o, cnt_o = _agg_kernel(h, esrc, eloc, ecnt)
        cnt = cnt_o.reshape(NPAD, 16)[:N, 0:1]
        h = _tc_layer(sum_o, sq_o, mx_o, mn_o, cnt, deg, h,
                      w, b.reshape(1, D))
    return h


# speedup vs baseline: 2.5459x; 1.0274x over previous
import functools

import jax
import jax.numpy as jnp
from jax import lax
from jax.experimental import pallas as pl
from jax.experimental.pallas import tpu as pltpu
from jax.experimental.pallas import tpu_sc as plsc
import numpy as np

N = 10000
E = 320000
D = 128
DELTA = float(np.log(2.0))

NC = 2
NS = 16
NW = NC * NS
R = 160
RW = 2 * R
NSEG = 2 * NW
NPAD = NSEG * R
C = 512
G = 128
NG = C // 16
NCHUNK = E // C
SEGCAP = E + G
HB = G // 2

_mesh = plsc.VectorSubcoreMesh(core_axis_name="c", subcore_axis_name="s")
_sc_params = pltpu.CompilerParams(needs_layout_passes=False)


def _worker_id():
    return lax.axis_index("s") * NC + lax.axis_index("c")


_build_out_type = (
    jax.ShapeDtypeStruct((NSEG * SEGCAP,), jnp.int32),
    jax.ShapeDtypeStruct((NSEG * SEGCAP,), jnp.int32),
    jax.ShapeDtypeStruct((NSEG * 16,), jnp.int32),
    jax.ShapeDtypeStruct((NW, RW, 16), jnp.float32),
)


@functools.partial(
    pl.kernel,
    out_type=_build_out_type,
    mesh=_mesh,
    scratch_types=[
        pltpu.VMEM((C,), jnp.int32),
        pltpu.VMEM((C,), jnp.int32),
        pltpu.VMEM((G + 32,), jnp.int32),
        pltpu.VMEM((G + 32,), jnp.int32),
        pltpu.VMEM((G + 32,), jnp.int32),
        pltpu.VMEM((G + 32,), jnp.int32),
        pltpu.VMEM((32,), jnp.int32),
        pltpu.VMEM((16,), jnp.int32),
        pltpu.VMEM((RW, 16), jnp.float32),
    ],
    compiler_params=_sc_params,
)
def _build_kernel(src_hbm, dst_hbm, esrc_o, eloc_o, ecnt_o, deg_o,
                  dst_v, srcc_v, ilo_v, llo_v, ihi_v, lhi_v,
                  tmp_v, cntb_v, dacc_v):
    wid = _worker_id()
    zero16 = jnp.zeros((16,), jnp.float32)
    lane = lax.iota(jnp.int32, 16)
    onehot = jnp.where(lane == 0, 1.0, 0.0)
    start = wid * RW

    def zero_body(i, _):
        dacc_v[i] = zero16
        return 0

    lax.fori_loop(0, RW, zero_body, 0)

    def append_and_flush(iv, lv, sv, locv, m, pend, nfl, seg):
        cs = plsc.cumsum(m.astype(jnp.int32))
        pos = jnp.where(m, pend + cs - 1, 0)
        plsc.store_scatter(iv.at[pl.ds(0, G + 32)], [pos], sv, mask=m)
        plsc.store_scatter(lv.at[pl.ds(0, G + 32)], [pos], locv, mask=m)
        pend = pend + cs[15]

        def flush(carry):
            pend, nfl = carry
            off = seg * SEGCAP + nfl * G
            pltpu.sync_copy(iv.at[pl.ds(0, G)], esrc_o.at[pl.ds(off, G)])
            pltpu.sync_copy(lv.at[pl.ds(0, G)], eloc_o.at[pl.ds(off, G)])
            rem = pend - G
            tmask = lane < rem
            ti = iv[pl.ds(G, 16)]
            tl = lv[pl.ds(G, 16)]
            plsc.store_scatter(iv.at[pl.ds(0, G + 32)], [lane], ti,
                               mask=tmask)
            plsc.store_scatter(lv.at[pl.ds(0, G + 32)], [lane], tl,
                               mask=tmask)
            return (rem, nfl + 1)

        return lax.cond(pend >= G, flush, lambda q: q, (pend, nfl))

    def chunk_body(c, carry):
        pltpu.sync_copy(dst_hbm.at[pl.ds(c * C, C)], dst_v)
        pltpu.sync_copy(src_hbm.at[pl.ds(c * C, C)], srcc_v)

        def group_body(g, carry):
            plo, nlo, phi, nhi = carry
            sv = srcc_v[pl.ds(g * 16, 16)]
            dv = dst_v[pl.ds(g * 16, 16)]

            ms = (sv >= start) & (sv < start + RW)
            css = plsc.cumsum(ms.astype(jnp.int32))
            poss = jnp.where(ms, css - 1, 0)
            plsc.store_scatter(tmp_v.at[pl.ds(0, 32)], [poss],
                               sv - start, mask=ms)
            ks = css[15]

            def sdeg_body(i, _):
                d = tmp_v[pl.ds(i, 16)][0]
                plsc.addupdate(dacc_v.at[d], onehot)
                return 0

            lax.fori_loop(0, ks, sdeg_body, 0)

            loc = dv - start
            md = (dv >= start) & (dv < start + RW)
            mlo = md & (loc < R)
            mhi = md & (loc >= R)
            plo, nlo = append_and_flush(ilo_v, llo_v, sv, loc, mlo,
                                        plo, nlo, 2 * wid)
            phi, nhi = append_and_flush(ihi_v, lhi_v, sv, loc - R, mhi,
                                        phi, nhi, 2 * wid + 1)
            return (plo, nlo, phi, nhi)

        return lax.fori_loop(0, NG, group_body, carry)

    plo, nlo, phi, nhi = lax.fori_loop(0, NCHUNK, chunk_body, (0, 0, 0, 0))

    def make_tail(iv, lv, seg):
        def tail(carry):
            pend, nfl = carry
            for i in range(G // 16):
                sl = pl.ds(16 * i, 16)
                mm = (lane + 16 * i) >= pend
                lv[sl] = jnp.where(mm, R, lv[sl])
            off = seg * SEGCAP + nfl * G
            pltpu.sync_copy(iv.at[pl.ds(0, G)], esrc_o.at[pl.ds(off, G)])
            pltpu.sync_copy(lv.at[pl.ds(0, G)], eloc_o.at[pl.ds(off, G)])
            return (0, nfl + 1)

        return tail

    _, nlo = lax.cond(plo > 0, make_tail(ilo_v, llo_v, 2 * wid),
                      lambda q: q, (plo, nlo))
    _, nhi = lax.cond(phi > 0, make_tail(ihi_v, lhi_v, 2 * wid + 1),
                      lambda q: q, (phi, nhi))

    cntb_v[pl.ds(0, 16)] = jnp.full((16,), 1, jnp.int32) * nlo
    pltpu.sync_copy(cntb_v, ecnt_o.at[pl.ds(2 * wid * 16, 16)])
    cntb_v[pl.ds(0, 16)] = jnp.full((16,), 1, jnp.int32) * nhi
    pltpu.sync_copy(cntb_v, ecnt_o.at[pl.ds((2 * wid + 1) * 16, 16)])
    pltpu.sync_copy(dacc_v, deg_o.at[wid])


_agg_out_type = (
    jax.ShapeDtypeStruct((NPAD, D), jnp.float32),
    jax.ShapeDtypeStruct((NPAD, D), jnp.float32),
    jax.ShapeDtypeStruct((NPAD, D), jnp.float32),
    jax.ShapeDtypeStruct((NPAD, D), jnp.float32),
    jax.ShapeDtypeStruct((NSEG, R, 16), jnp.float32),
)


@functools.partial(
    pl.kernel,
    out_type=_agg_out_type,
    mesh=_mesh,
    scratch_types=[
        pltpu.VMEM((HB,), jnp.int32),
        pltpu.VMEM((HB,), jnp.int32),
        pltpu.VMEM((HB + 16,), jnp.int32),
        pltpu.VMEM((HB + 16,), jnp.int32),
        pltpu.VMEM((HB, D), jnp.float32),
        pltpu.VMEM((HB, D), jnp.float32),
        pltpu.VMEM((R + 1, D), jnp.float32),
        pltpu.VMEM((R + 1, D), jnp.float32),
        pltpu.VMEM((R + 1, D), jnp.float32),
        pltpu.VMEM((R + 1, D), jnp.float32),
        pltpu.VMEM((R + 1, 16), jnp.float32),
        pltpu.VMEM((16,), jnp.int32),
        pltpu.SemaphoreType.DMA,
        pltpu.SemaphoreType.DMA,
    ],
    compiler_params=_sc_params,
)
def _agg_kernel(h_hbm, esrc_hbm, eloc_hbm, ecnt_hbm,
                sum_o, sq_o, mx_o, mn_o, cnt_o,
                gidx0_v, gidx1_v, loc0_v, loc1_v, rows0_v, rows1_v,
                sacc, qacc, xacc, nacc, cacc,
                cnt_s, sem0, sem1):
    wid = _worker_id()
    zero16 = jnp.zeros((16,), jnp.float32)
    ninf16 = jnp.full((16,), -jnp.inf, jnp.float32)
    pinf16 = jnp.full((16,), jnp.inf, jnp.float32)
    lane = lax.iota(jnp.int32, 16)
    onehot = jnp.where(lane == 0, 1.0, 0.0)

    def seg_body(sp, _):
        seg = 2 * wid + sp
        base = seg * SEGCAP
        pltpu.sync_copy(ecnt_hbm.at[pl.ds(seg * 16, 16)], cnt_s)
        ng = cnt_s[pl.ds(0, 16)][0]

        def zero_body(i, _):
            for j in range(D // 16):
                sl = pl.ds(16 * j, 16)
                sacc[i, sl] = zero16
                qacc[i, sl] = zero16
                xacc[i, sl] = ninf16
                nacc[i, sl] = pinf16
            cacc[i] = zero16
            return 0

        lax.fori_loop(0, R + 1, zero_body, 0)

        bufs = ((gidx0_v, loc0_v, rows0_v, sem0),
                (gidx1_v, loc1_v, rows1_v, sem1))
        nh = 2 * ng

        def prefetch(h, buf):
            gidx_v, loc_v, rows_v, sem = bufs[buf]

            def start(_):
                off = base + h * HB
                pltpu.sync_copy(esrc_hbm.at[pl.ds(off, HB)], gidx_v)
                pltpu.sync_copy(eloc_hbm.at[pl.ds(off, HB)],
                                loc_v.at[pl.ds(0, HB)])
                pltpu.async_copy(h_hbm.at[gidx_v], rows_v, sem)
                return 0

            lax.cond(h < nh, start, lambda q: 0, 0)

        def consume(h, buf):
            gidx_v, loc_v, rows_v, sem = bufs[buf]

            def work(_):
                prefetch(h + 1, 1 - buf)
                pltpu.make_async_copy(h_hbm.at[gidx_v], rows_v, sem).wait()

                def edge_body(e, _):
                    d = loc_v[pl.ds(e, 16)][0]
                    plsc.addupdate(cacc.at[d], onehot)
                    for j in range(D // 16):
                        sl = pl.ds(16 * j, 16)
                        m = rows_v[e, sl]
                        plsc.addupdate(sacc.at[d, sl], m)
                        plsc.addupdate(qacc.at[d, sl], m * m)
                        xacc[d, sl] = jnp.maximum(xacc[d, sl], m)
                        nacc[d, sl] = jnp.minimum(nacc[d, sl], m)
                    return 0

                lax.fori_loop(0, HB, edge_body, 0)
                return 0

            lax.cond(h < nh, work, lambda q: 0, 0)

        prefetch(0, 0)

        def pair_body(p, _):
            consume(2 * p, 0)
            consume(2 * p + 1, 1)
            return 0

        lax.fori_loop(0, ng, pair_body, 0)

        row0 = pl.ds(seg * R, R)
        pltpu.sync_copy(sacc.at[pl.ds(0, R)], sum_o.at[row0])
        pltpu.sync_copy(qacc.at[pl.ds(0, R)], sq_o.at[row0])
        pltpu.sync_copy(xacc.at[pl.ds(0, R)], mx_o.at[row0])
        pltpu.sync_copy(nacc.at[pl.ds(0, R)], mn_o.at[row0])
        pltpu.sync_copy(cacc.at[pl.ds(0, R)], cnt_o.at[seg])
        return 0

    lax.fori_loop(0, 2, seg_body, 0)


RB = 1000


def _tc_body(sum_r, sq_r, mx_r, mn_r, cnt_r, deg_r, h_r, w_r, b_r, out_r):
    cnt = cnt_r[...]
    cntc = jnp.maximum(cnt, 1.0)
    s = sum_r[...]
    mean = s / cntc
    var = jnp.maximum(sq_r[...] / cntc - mean * mean, 0.0)
    std = jnp.sqrt(var + 1e-5)
    pos = cnt > 0.0
    mx = jnp.where(pos, mx_r[...], 0.0)
    mn = jnp.where(pos, mn_r[...], 0.0)
    agg = jnp.concatenate([mean, mx, mn, std], axis=1)
    wa = w_r[0:4 * D, :]
    wb = w_r[4 * D:8 * D, :]
    wc = w_r[8 * D:12 * D, :]
    b1 = jnp.dot(agg, wa, preferred_element_type=jnp.float32)
    b2 = jnp.dot(agg, wb, preferred_element_type=jnp.float32)
    b3 = jnp.dot(agg, wc, preferred_element_type=jnp.float32)
    degc = jnp.maximum(deg_r[...], 1.0)
    logd = jnp.log(degc + 1.0)
    amp = logd * (1.0 / DELTA)
    att = DELTA / logd
    out_r[...] = b1 + amp * b2 + att * b3 + b_r[...] + h_r[...]


def _tc_layer(sum_a, sq_a, mx_a, mn_a, cnt_a, deg_a, h, w, b):
    grid = (N // RB,)
    row_spec = pl.BlockSpec((RB, D), lambda i: (i, 0))
    col_spec = pl.BlockSpec((RB, 1), lambda i: (i, 0))
    return pl.pallas_call(
        _tc_body,
        grid=grid,
        in_specs=[
            row_spec, row_spec, row_spec, row_spec,
            col_spec, col_spec, row_spec,
            pl.BlockSpec((12 * D, D), lambda i: (0, 0)),
            pl.BlockSpec((1, D), lambda i: (0, 0)),
        ],
        out_specs=row_spec,
        out_shape=jax.ShapeDtypeStruct((N, D), jnp.float32),
    )(sum_a, sq_a, mx_a, mn_a, cnt_a, deg_a, h, w, b)


def kernel(x, edge_index, W0, b0, W1, b1, W2, b2):
    src = edge_index[0]
    dst = edge_index[1]
    esrc, eloc, ecnt, deg_o = _build_kernel(src, dst)
    deg = deg_o.reshape(NPAD, 16)[:N, 0:1]
    h = x
    for w, b in ((W0, b0), (W1, b1), (W2, b2)):
        sum_o, sq_o, mx_o, mn_o, cnt_o = _agg_kernel(h, esrc, eloc, ecnt)
        cnt = cnt_o.reshape(NPAD, 16)[:N, 0:1]
        h = _tc_layer(sum_o, sq_o, mx_o, mn_o, cnt, deg, h,
                      w, b.reshape(1, D))
    return h


# speedup vs baseline: 3.0168x; 1.1850x over previous
import functools

import jax
import jax.numpy as jnp
from jax import lax
from jax.experimental import pallas as pl
from jax.experimental.pallas import tpu as pltpu
from jax.experimental.pallas import tpu_sc as plsc
import numpy as np

N = 10000
E = 320000
D = 128
DELTA = float(np.log(2.0))

NC = 2
NS = 16
NW = NC * NS
R = 160
RW = 2 * R
NSEG = 2 * NW
NPAD = NSEG * R
C = 512
G = 128
NG = C // 16
NCHUNK = E // C
SEGCAP = E + G
HB = G // 2

_mesh = plsc.VectorSubcoreMesh(core_axis_name="c", subcore_axis_name="s")
_sc_params = pltpu.CompilerParams(needs_layout_passes=False)


def _worker_id():
    return lax.axis_index("s") * NC + lax.axis_index("c")


_build_out_type = (
    jax.ShapeDtypeStruct((NSEG * SEGCAP,), jnp.int32),
    jax.ShapeDtypeStruct((NSEG * SEGCAP,), jnp.int32),
    jax.ShapeDtypeStruct((NSEG * 16,), jnp.int32),
    jax.ShapeDtypeStruct((NW * RW * 16,), jnp.float32),
)


@functools.partial(
    pl.kernel,
    out_type=_build_out_type,
    mesh=_mesh,
    scratch_types=[
        pltpu.VMEM((C,), jnp.int32),
        pltpu.VMEM((C,), jnp.int32),
        pltpu.VMEM((G + 32,), jnp.int32),
        pltpu.VMEM((G + 32,), jnp.int32),
        pltpu.VMEM((G + 32,), jnp.int32),
        pltpu.VMEM((G + 32,), jnp.int32),
        pltpu.VMEM((16,), jnp.int32),
        pltpu.VMEM((RW * 16,), jnp.float32),
    ],
    compiler_params=_sc_params,
)
def _build_kernel(src_hbm, dst_hbm, esrc_o, eloc_o, ecnt_o, deg_o,
                  dst_v, srcc_v, ilo_v, llo_v, ihi_v, lhi_v,
                  cntb_v, dacc_v):
    wid = _worker_id()
    zero16 = jnp.zeros((16,), jnp.float32)
    lane = lax.iota(jnp.int32, 16)
    ones16 = jnp.full((16,), 1.0, jnp.float32)
    start = wid * RW

    def zero_body(i, _):
        dacc_v[pl.ds(i * 16, 16)] = zero16
        return 0

    lax.fori_loop(0, RW, zero_body, 0)

    def append_and_flush(iv, lv, sv, locv, m, pend, nfl, seg):
        cs = plsc.cumsum(m.astype(jnp.int32))
        pos = jnp.where(m, pend + cs - 1, 0)
        plsc.store_scatter(iv.at[pl.ds(0, G + 32)], [pos], sv, mask=m)
        plsc.store_scatter(lv.at[pl.ds(0, G + 32)], [pos], locv, mask=m)
        pend = pend + cs[15]

        def flush(carry):
            pend, nfl = carry
            off = seg * SEGCAP + nfl * G
            pltpu.sync_copy(iv.at[pl.ds(0, G)], esrc_o.at[pl.ds(off, G)])
            pltpu.sync_copy(lv.at[pl.ds(0, G)], eloc_o.at[pl.ds(off, G)])
            rem = pend - G
            tmask = lane < rem
            ti = iv[pl.ds(G, 16)]
            tl = lv[pl.ds(G, 16)]
            plsc.store_scatter(iv.at[pl.ds(0, G + 32)], [lane], ti,
                               mask=tmask)
            plsc.store_scatter(lv.at[pl.ds(0, G + 32)], [lane], tl,
                               mask=tmask)
            return (rem, nfl + 1)

        return lax.cond(pend >= G, flush, lambda q: q, (pend, nfl))

    def chunk_body(c, carry):
        pltpu.sync_copy(dst_hbm.at[pl.ds(c * C, C)], dst_v)
        pltpu.sync_copy(src_hbm.at[pl.ds(c * C, C)], srcc_v)

        def group_body(g, carry):
            plo, nlo, phi, nhi = carry
            sv = srcc_v[pl.ds(g * 16, 16)]
            dv = dst_v[pl.ds(g * 16, 16)]

            ms = (sv >= start) & (sv < start + RW)
            sloc = jnp.where(ms, (sv - start) * 16, 0)
            plsc.addupdate_scatter(dacc_v.at[pl.ds(0, RW * 16)], [sloc],
                                   ones16, mask=ms)

            loc = dv - start
            md = (dv >= start) & (dv < start + RW)
            mlo = md & (loc < R)
            mhi = md & (loc >= R)
            plo, nlo = append_and_flush(ilo_v, llo_v, sv, loc, mlo,
                                        plo, nlo, 2 * wid)
            phi, nhi = append_and_flush(ihi_v, lhi_v, sv, loc - R, mhi,
                                        phi, nhi, 2 * wid + 1)
            return (plo, nlo, phi, nhi)

        return lax.fori_loop(0, NG, group_body, carry)

    plo, nlo, phi, nhi = lax.fori_loop(0, NCHUNK, chunk_body, (0, 0, 0, 0))

    def make_tail(iv, lv, seg):
        def tail(carry):
            pend, nfl = carry
            for i in range(G // 16):
                sl = pl.ds(16 * i, 16)
                mm = (lane + 16 * i) >= pend
                lv[sl] = jnp.where(mm, R, lv[sl])
            off = seg * SEGCAP + nfl * G
            pltpu.sync_copy(iv.at[pl.ds(0, G)], esrc_o.at[pl.ds(off, G)])
            pltpu.sync_copy(lv.at[pl.ds(0, G)], eloc_o.at[pl.ds(off, G)])
            return (0, nfl + 1)

        return tail

    _, nlo = lax.cond(plo > 0, make_tail(ilo_v, llo_v, 2 * wid),
                      lambda q: q, (plo, nlo))
    _, nhi = lax.cond(phi > 0, make_tail(ihi_v, lhi_v, 2 * wid + 1),
                      lambda q: q, (phi, nhi))

    cntb_v[pl.ds(0, 16)] = jnp.full((16,), 1, jnp.int32) * nlo
    pltpu.sync_copy(cntb_v, ecnt_o.at[pl.ds(2 * wid * 16, 16)])
    cntb_v[pl.ds(0, 16)] = jnp.full((16,), 1, jnp.int32) * nhi
    pltpu.sync_copy(cntb_v, ecnt_o.at[pl.ds((2 * wid + 1) * 16, 16)])
    pltpu.sync_copy(dacc_v, deg_o.at[pl.ds(wid * RW * 16, RW * 16)])


_agg_out_type = (
    jax.ShapeDtypeStruct((NPAD, D), jnp.float32),
    jax.ShapeDtypeStruct((NPAD, D), jnp.float32),
    jax.ShapeDtypeStruct((NPAD, D), jnp.float32),
    jax.ShapeDtypeStruct((NPAD, D), jnp.float32),
    jax.ShapeDtypeStruct((NSEG, R, 16), jnp.float32),
)


@functools.partial(
    pl.kernel,
    out_type=_agg_out_type,
    mesh=_mesh,
    scratch_types=[
        pltpu.VMEM((HB,), jnp.int32),
        pltpu.VMEM((HB,), jnp.int32),
        pltpu.VMEM((HB + 16,), jnp.int32),
        pltpu.VMEM((HB + 16,), jnp.int32),
        pltpu.VMEM((HB, D), jnp.float32),
        pltpu.VMEM((HB, D), jnp.float32),
        pltpu.VMEM((R + 1, D), jnp.float32),
        pltpu.VMEM((R + 1, D), jnp.float32),
        pltpu.VMEM((R + 1, D), jnp.float32),
        pltpu.VMEM((R + 1, D), jnp.float32),
        pltpu.VMEM((R + 1, 16), jnp.float32),
        pltpu.VMEM((16,), jnp.int32),
        pltpu.SemaphoreType.DMA,
        pltpu.SemaphoreType.DMA,
    ],
    compiler_params=_sc_params,
)
def _agg_kernel(h_hbm, esrc_hbm, eloc_hbm, ecnt_hbm,
                sum_o, sq_o, mx_o, mn_o, cnt_o,
                gidx0_v, gidx1_v, loc0_v, loc1_v, rows0_v, rows1_v,
                sacc, qacc, xacc, nacc, cacc,
                cnt_s, sem0, sem1):
    wid = _worker_id()
    zero16 = jnp.zeros((16,), jnp.float32)
    ninf16 = jnp.full((16,), -jnp.inf, jnp.float32)
    pinf16 = jnp.full((16,), jnp.inf, jnp.float32)
    lane = lax.iota(jnp.int32, 16)
    onehot = jnp.where(lane == 0, 1.0, 0.0)

    def seg_body(sp, _):
        seg = 2 * wid + sp
        base = seg * SEGCAP
        pltpu.sync_copy(ecnt_hbm.at[pl.ds(seg * 16, 16)], cnt_s)
        ng = cnt_s[pl.ds(0, 16)][0]

        def zero_body(i, _):
            for j in range(D // 16):
                sl = pl.ds(16 * j, 16)
                sacc[i, sl] = zero16
                qacc[i, sl] = zero16
                xacc[i, sl] = ninf16
                nacc[i, sl] = pinf16
            cacc[i] = zero16
            return 0

        lax.fori_loop(0, R + 1, zero_body, 0)

        bufs = ((gidx0_v, loc0_v, rows0_v, sem0),
                (gidx1_v, loc1_v, rows1_v, sem1))
        nh = 2 * ng

        def prefetch(h, buf):
            gidx_v, loc_v, rows_v, sem = bufs[buf]

            def start(_):
                off = base + h * HB
                pltpu.sync_copy(esrc_hbm.at[pl.ds(off, HB)], gidx_v)
                pltpu.sync_copy(eloc_hbm.at[pl.ds(off, HB)],
                                loc_v.at[pl.ds(0, HB)])
                pltpu.async_copy(h_hbm.at[gidx_v], rows_v, sem)
                return 0

            lax.cond(h < nh, start, lambda q: 0, 0)

        def consume(h, buf):
            gidx_v, loc_v, rows_v, sem = bufs[buf]

            def work(_):
                prefetch(h + 1, 1 - buf)
                pltpu.make_async_copy(h_hbm.at[gidx_v], rows_v, sem).wait()

                def edge_body(e, _):
                    d = loc_v[pl.ds(e, 16)][0]
                    plsc.addupdate(cacc.at[d], onehot)
                    for j in range(D // 16):
                        sl = pl.ds(16 * j, 16)
                        m = rows_v[e, sl]
                        plsc.addupdate(sacc.at[d, sl], m)
                        plsc.addupdate(qacc.at[d, sl], m * m)
                        xacc[d, sl] = jnp.maximum(xacc[d, sl], m)
                        nacc[d, sl] = jnp.minimum(nacc[d, sl], m)
                    return 0

                lax.fori_loop(0, HB, edge_body, 0)
                return 0

            lax.cond(h < nh, work, lambda q: 0, 0)

        prefetch(0, 0)

        def pair_body(p, _):
            consume(2 * p, 0)
            consume(2 * p + 1, 1)
            return 0

        lax.fori_loop(0, ng, pair_body, 0)

        row0 = pl.ds(seg * R, R)
        pltpu.sync_copy(sacc.at[pl.ds(0, R)], sum_o.at[row0])
        pltpu.sync_copy(qacc.at[pl.ds(0, R)], sq_o.at[row0])
        pltpu.sync_copy(xacc.at[pl.ds(0, R)], mx_o.at[row0])
        pltpu.sync_copy(nacc.at[pl.ds(0, R)], mn_o.at[row0])
        pltpu.sync_copy(cacc.at[pl.ds(0, R)], cnt_o.at[seg])
        return 0

    lax.fori_loop(0, 2, seg_body, 0)


RB = 1000


def _tc_body(sum_r, sq_r, mx_r, mn_r, cnt_r, deg_r, h_r, w_r, b_r, out_r):
    cnt = cnt_r[...]
    cntc = jnp.maximum(cnt, 1.0)
    s = sum_r[...]
    mean = s / cntc
    var = jnp.maximum(sq_r[...] / cntc - mean * mean, 0.0)
    std = jnp.sqrt(var + 1e-5)
    pos = cnt > 0.0
    mx = jnp.where(pos, mx_r[...], 0.0)
    mn = jnp.where(pos, mn_r[...], 0.0)
    agg = jnp.concatenate([mean, mx, mn, std], axis=1)
    wa = w_r[0:4 * D, :]
    wb = w_r[4 * D:8 * D, :]
    wc = w_r[8 * D:12 * D, :]
    b1 = jnp.dot(agg, wa, preferred_element_type=jnp.float32)
    b2 = jnp.dot(agg, wb, preferred_element_type=jnp.float32)
    b3 = jnp.dot(agg, wc, preferred_element_type=jnp.float32)
    degc = jnp.maximum(deg_r[...], 1.0)
    logd = jnp.log(degc + 1.0)
    amp = logd * (1.0 / DELTA)
    att = DELTA / logd
    out_r[...] = b1 + amp * b2 + att * b3 + b_r[...] + h_r[...]


def _tc_layer(sum_a, sq_a, mx_a, mn_a, cnt_a, deg_a, h, w, b):
    grid = (N // RB,)
    row_spec = pl.BlockSpec((RB, D), lambda i: (i, 0))
    col_spec = pl.BlockSpec((RB, 1), lambda i: (i, 0))
    return pl.pallas_call(
        _tc_body,
        grid=grid,
        in_specs=[
            row_spec, row_spec, row_spec, row_spec,
            col_spec, col_spec, row_spec,
            pl.BlockSpec((12 * D, D), lambda i: (0, 0)),
            pl.BlockSpec((1, D), lambda i: (0, 0)),
        ],
        out_specs=row_spec,
        out_shape=jax.ShapeDtypeStruct((N, D), jnp.float32),
    )(sum_a, sq_a, mx_a, mn_a, cnt_a, deg_a, h, w, b)


def kernel(x, edge_index, W0, b0, W1, b1, W2, b2):
    src = edge_index[0]
    dst = edge_index[1]
    esrc, eloc, ecnt, deg_o = _build_kernel(src, dst)
    deg = deg_o.reshape(NPAD, 16)[:N, 0:1]
    h = x
    for w, b in ((W0, b0), (W1, b1), (W2, b2)):
        sum_o, sq_o, mx_o, mn_o, cnt_o = _agg_kernel(h, esrc, eloc, ecnt)
        cnt = cnt_o.reshape(NPAD, 16)[:N, 0:1]
        h = _tc_layer(sum_o, sq_o, mx_o, mn_o, cnt, deg, h,
                      w, b.reshape(1, D))
    return h
